# baseline (speedup 1.0000x reference)
import functools

import jax
import jax.numpy as jnp
from jax import lax
from jax.experimental import pallas as pl
from jax.experimental.pallas import tpu as pltpu
from jax.experimental.pallas import tpu_sc as plsc

_B = 16384
_N = 1000
_WS = 50
_ALPHA = 0.1
_EXPF = 2.0

_NC = 2
_NS = 16
_NW = _NC * _NS
_GROWS = _B * _WS // 128
_RW = _GROWS // _NW


_PW = _B * _WS // _NW
_RPW = _B // _NW
_PWP = _PW + 16


def _sc_gather_body(t_hbm, es_hbm, pr_hbm, esg_hbm, prg_hbm, mk_hbm,
                    t_v, idx_v, mk_v, ge_v, gp_v, buf, sem):
    c = lax.axis_index("c")
    s = lax.axis_index("s")
    wid = s * _NC + c
    base = wid * _PW
    row0 = wid * _RPW
    pltpu.sync_copy(t_hbm.at[pl.ds(base, _PW)], t_v.at[pl.ds(0, _PW)])
    lanes = lax.iota(jnp.int32, 16)
    one = jnp.full((16,), 1.0, jnp.float32)
    zero = jnp.full((16,), 0.0, jnp.float32)

    def row_step(r, carry):
        rowbase = (row0 + r) * _N
        for k in range(4):
            o = r * _WS + k * 16
            t = t_v[pl.ds(o, 16)]
            wv = lanes + k * 16
            m = wv < _WS
            a = jnp.where(t > 0, t - 1, _N - 1)
            a = jnp.clip(a, 0, _N - 1)
            plsc.store_scatter(buf, [a], wv, mask=m)
            idx_v[pl.ds(o, 16)] = rowbase + a
        for k in range(4):
            o = r * _WS + k * 16
            t = t_v[pl.ds(o, 16)]
            wv = lanes + k * 16
            m = wv < _WS
            a = jnp.where(t > 0, t - 1, _N - 1)
            a = jnp.clip(a, 0, _N - 1)
            g = plsc.load_gather(buf, [a], mask=m)
            keep = (g == wv) & (t > 0) & m
            mk_v[pl.ds(o, 16)] = jnp.where(keep, one, zero)
        return carry

    lax.fori_loop(0, _RPW, row_step, 0)
    idx_v[pl.ds(_PW, 16)] = jnp.full((16,), 0, jnp.int32)
    pltpu.async_copy(es_hbm.at[idx_v], ge_v, sem).wait()
    pltpu.async_copy(pr_hbm.at[idx_v], gp_v, sem).wait()
    pltpu.sync_copy(ge_v.at[pl.ds(0, _PW)], esg_hbm.at[pl.ds(base, _PW)])
    pltpu.sync_copy(gp_v.at[pl.ds(0, _PW)], prg_hbm.at[pl.ds(base, _PW)])
    pltpu.sync_copy(mk_v.at[pl.ds(0, _PW)], mk_hbm.at[pl.ds(base, _PW)])


def _sc_gather(t_flat, es_flat, pr_flat):
    f = functools.partial(
        pl.kernel,
        out_type=[jax.ShapeDtypeStruct((_B * _WS,), jnp.float32),
                  jax.ShapeDtypeStruct((_B * _WS,), jnp.float32),
                  jax.ShapeDtypeStruct((_B * _WS,), jnp.float32)],
        mesh=plsc.VectorSubcoreMesh(core_axis_name="c", subcore_axis_name="s"),
        compiler_params=pltpu.CompilerParams(needs_layout_passes=False),
        scratch_types=[
            pltpu.VMEM((_PWP,), jnp.int32),
            pltpu.VMEM((_PWP,), jnp.int32),
            pltpu.VMEM((_PWP,), jnp.float32),
            pltpu.VMEM((_PWP,), jnp.float32),
            pltpu.VMEM((_PWP,), jnp.float32),
            pltpu.VMEM((_N,), jnp.int32),
            pltpu.SemaphoreType.DMA,
        ],
    )(_sc_gather_body)
    return f(t_flat, es_flat, pr_flat)


_RD = 512


def _dense_body(es_ref, pr_ref, t_ref, ge_ref, gp_ref, mk_ref, out_ref):
    es = es_ref[...]
    pr = pr_ref[...]
    l1m = jnp.maximum(jnp.log1p(-es), -100.0)
    e2 = jnp.exp(_EXPF * es)
    rmax = jnp.max(pr, axis=1, keepdims=True)
    sexp = jnp.sum(jnp.exp(pr - rmax), axis=1)
    lse = jnp.sum(rmax[:, 0] + jnp.log(sexp))
    t = t_ref[...]
    ge = ge_ref[...]
    gp = gp_ref[...]
    mk = mk_ref[...]
    gl1m = jnp.maximum(jnp.log1p(-ge), -100.0)
    glp = jnp.maximum(jnp.log(ge), -100.0)
    corr = jnp.sum(mk * (gl1m - glp))
    wio = lax.broadcasted_iota(jnp.int32, t.shape, 1)
    wsel = jnp.min(jnp.where(t > 0, wio, _WS - 1), axis=1, keepdims=True)
    psel = jnp.sum(jnp.where(wio == wsel, gp, 0.0))
    lane = lax.broadcasted_iota(jnp.int32, (1, 1, 128), 2)
    out_ref[...] = jnp.where(lane == 0, jnp.sum(l1m),
                   jnp.where(lane == 1, jnp.sum(e2),
                   jnp.where(lane == 2, lse,
                   jnp.where(lane == 3, corr,
                   jnp.where(lane == 4, psel, 0.0)))))


def _dense(es, pr, targets, esg, prg, mk):
    g = _B // _RD
    return pl.pallas_call(
        _dense_body,
        grid=(g,),
        in_specs=[pl.BlockSpec((_RD, _N), lambda i: (i, 0)),
                  pl.BlockSpec((_RD, _N), lambda i: (i, 0)),
                  pl.BlockSpec((_RD, _WS), lambda i: (i, 0)),
                  pl.BlockSpec((_RD, _WS), lambda i: (i, 0)),
                  pl.BlockSpec((_RD, _WS), lambda i: (i, 0)),
                  pl.BlockSpec((_RD, _WS), lambda i: (i, 0))],
        out_specs=pl.BlockSpec((1, 1, 128), lambda i: (i, 0, 0)),
        out_shape=jax.ShapeDtypeStruct((g, 1, 128), jnp.float32),
    )(es, pr, targets, esg, prg, mk)


def kernel(enable_state, prediction, targets):
    t_flat = targets.reshape(-1)
    es_flat = enable_state.reshape(-1)
    pr_flat = prediction.reshape(-1)
    esg, prg, mk = _sc_gather(t_flat, es_flat, pr_flat)
    dense = _dense(enable_state, prediction, targets,
                   esg.reshape(_B, _WS), prg.reshape(_B, _WS),
                   mk.reshape(_B, _WS))
    s_l1m = jnp.sum(dense[:, 0, 0])
    s_e2 = jnp.sum(dense[:, 0, 1])
    s_lse = jnp.sum(dense[:, 0, 2])
    corr = jnp.sum(dense[:, 0, 3])
    psel = jnp.sum(dense[:, 0, 4])
    m = _B * _N
    loss1 = (corr - s_l1m) / m
    w1 = loss1 * (s_e2 / m)
    o1 = _ALPHA * w1
    loss2 = (s_lse - psel) / _B
    return (o1, loss2, o1 + loss2)


# speedup vs baseline: 1.1280x; 1.1280x over previous
import functools

import jax
import jax.numpy as jnp
from jax import lax
from jax.experimental import pallas as pl
from jax.experimental.pallas import tpu as pltpu
from jax.experimental.pallas import tpu_sc as plsc

_B = 16384
_N = 1000
_WS = 50
_ALPHA = 0.1
_EXPF = 2.0

_NC = 2
_NS = 16
_NW = _NC * _NS
_GROWS = _B * _WS // 128
_RW = _GROWS // _NW


_PW = _B * _WS // _NW
_RPW = _B // _NW
_PWP = _PW + 16


def _sc_gather_body(t_hbm, es_hbm, pr_hbm, esg_hbm, prg_hbm, mk_hbm,
                    t_v, idx_v, mk_v, ge_v, gp_v, buf, sem, sem2):
    c = lax.axis_index("c")
    s = lax.axis_index("s")
    wid = s * _NC + c
    base = wid * _PW
    row0 = wid * _RPW
    pltpu.sync_copy(t_hbm.at[pl.ds(base, _PW)], t_v.at[pl.ds(0, _PW)])
    lanes = lax.iota(jnp.int32, 16)
    one = jnp.full((16,), 1.0, jnp.float32)
    zero = jnp.full((16,), 0.0, jnp.float32)

    def row_step(r, carry):
        rowbase = (row0 + r) * _N
        for k in range(4):
            o = r * _WS + k * 16
            t = t_v[pl.ds(o, 16)]
            wv = lanes + k * 16
            m = wv < _WS
            a = jnp.where(t > 0, t - 1, _N - 1)
            a = jnp.clip(a, 0, _N - 1)
            plsc.store_scatter(buf, [a], wv, mask=m)
            idx_v[pl.ds(o, 16)] = rowbase + a
        for k in range(4):
            o = r * _WS + k * 16
            t = t_v[pl.ds(o, 16)]
            wv = lanes + k * 16
            m = wv < _WS
            a = jnp.where(t > 0, t - 1, _N - 1)
            a = jnp.clip(a, 0, _N - 1)
            g = plsc.load_gather(buf, [a], mask=m)
            keep = (g == wv) & (t > 0) & m
            mk_v[pl.ds(o, 16)] = jnp.where(keep, one, zero)
        return carry

    lax.fori_loop(0, _RPW, row_step, 0)
    idx_v[pl.ds(_PW, 16)] = jnp.full((16,), 0, jnp.int32)
    cpe = pltpu.make_async_copy(es_hbm.at[idx_v], ge_v, sem)
    cpp = pltpu.make_async_copy(pr_hbm.at[idx_v], gp_v, sem2)
    cpe.start()
    cpp.start()
    cpe.wait()
    cpp.wait()
    pltpu.sync_copy(ge_v.at[pl.ds(0, _PW)], esg_hbm.at[pl.ds(base, _PW)])
    pltpu.sync_copy(gp_v.at[pl.ds(0, _PW)], prg_hbm.at[pl.ds(base, _PW)])
    pltpu.sync_copy(mk_v.at[pl.ds(0, _PW)], mk_hbm.at[pl.ds(base, _PW)])


def _sc_gather(t_flat, es_flat, pr_flat):
    f = functools.partial(
        pl.kernel,
        out_type=[jax.ShapeDtypeStruct((_B * _WS,), jnp.float32),
                  jax.ShapeDtypeStruct((_B * _WS,), jnp.float32),
                  jax.ShapeDtypeStruct((_B * _WS,), jnp.float32)],
        mesh=plsc.VectorSubcoreMesh(core_axis_name="c", subcore_axis_name="s"),
        compiler_params=pltpu.CompilerParams(needs_layout_passes=False),
        scratch_types=[
            pltpu.VMEM((_PWP,), jnp.int32),
            pltpu.VMEM((_PWP,), jnp.int32),
            pltpu.VMEM((_PWP,), jnp.float32),
            pltpu.VMEM((_PWP,), jnp.float32),
            pltpu.VMEM((_PWP,), jnp.float32),
            pltpu.VMEM((_N,), jnp.int32),
            pltpu.SemaphoreType.DMA,
            pltpu.SemaphoreType.DMA,
        ],
    )(_sc_gather_body)
    return f(t_flat, es_flat, pr_flat)


_RD = 512


def _dense_body(es_ref, pr_ref, out_ref):
    es = es_ref[...]
    pr = pr_ref[...]
    l1m = jnp.maximum(jnp.log1p(-es), -100.0)
    e2 = jnp.exp(_EXPF * es)
    rmax = jnp.max(pr, axis=1, keepdims=True)
    sexp = jnp.sum(jnp.exp(pr - rmax), axis=1)
    lse = jnp.sum(rmax[:, 0] + jnp.log(sexp))
    lane = lax.broadcasted_iota(jnp.int32, (1, 1, 128), 2)
    out_ref[...] = jnp.where(lane == 0, jnp.sum(l1m),
                   jnp.where(lane == 1, jnp.sum(e2),
                   jnp.where(lane == 2, lse, 0.0)))


def _dense(es, pr):
    g = _B // _RD
    return pl.pallas_call(
        _dense_body,
        grid=(g,),
        in_specs=[pl.BlockSpec((_RD, _N), lambda i: (i, 0)),
                  pl.BlockSpec((_RD, _N), lambda i: (i, 0))],
        out_specs=pl.BlockSpec((1, 1, 128), lambda i: (i, 0, 0)),
        out_shape=jax.ShapeDtypeStruct((g, 1, 128), jnp.float32),
    )(es, pr)


def _corr_body(t_ref, ge_ref, gp_ref, mk_ref, dp_ref, o1_ref, o2_ref, o3_ref):
    t = t_ref[...]
    ge = ge_ref[...]
    gp = gp_ref[...]
    mk = mk_ref[...]
    l1m = jnp.maximum(jnp.log1p(-ge), -100.0)
    lp = jnp.maximum(jnp.log(ge), -100.0)
    corr = jnp.sum(mk * (l1m - lp))
    wio = lax.broadcasted_iota(jnp.int32, t.shape, 1)
    wsel = jnp.min(jnp.where(t > 0, wio, _WS - 1), axis=1, keepdims=True)
    psel = jnp.sum(jnp.where(wio == wsel, gp, 0.0))
    dp = dp_ref[...]
    s_l1m = jnp.sum(dp[:, 0, 0])
    s_e2 = jnp.sum(dp[:, 0, 1])
    s_lse = jnp.sum(dp[:, 0, 2])
    m = jnp.float32(_B * _N)
    loss1 = (corr - s_l1m) / m
    o1 = _ALPHA * (loss1 * (s_e2 / m))
    loss2 = (s_lse - psel) / _B
    o1_ref[0] = o1
    o2_ref[0] = loss2
    o3_ref[0] = o1 + loss2


def _corr(targets, esg, prg, mk, dpart):
    sds = jax.ShapeDtypeStruct((1,), jnp.float32)
    return pl.pallas_call(
        _corr_body,
        in_specs=[pl.BlockSpec(memory_space=pltpu.VMEM)] * 5,
        out_specs=[pl.BlockSpec(memory_space=pltpu.SMEM)] * 3,
        out_shape=[sds, sds, sds],
    )(targets, esg, prg, mk, dpart)


def kernel(enable_state, prediction, targets):
    t_flat = targets.reshape(-1)
    es_flat = enable_state.reshape(-1)
    pr_flat = prediction.reshape(-1)
    esg, prg, mk = _sc_gather(t_flat, es_flat, pr_flat)
    dense = _dense(enable_state, prediction)
    o1, l2, o3 = _corr(targets, esg.reshape(_B, _WS), prg.reshape(_B, _WS),
                       mk.reshape(_B, _WS), dense)
    return (o1[0], l2[0], o3[0])


# speedup vs baseline: 1.6306x; 1.4456x over previous
import functools

import jax
import jax.numpy as jnp
from jax import lax
from jax.experimental import pallas as pl
from jax.experimental.pallas import tpu as pltpu
from jax.experimental.pallas import tpu_sc as plsc

_B = 16384
_N = 1000
_WS = 50
_ALPHA = 0.1
_EXPF = 2.0

_NC = 2
_NS = 16
_NW = _NC * _NS
_GROWS = _B * _WS // 128
_RW = _GROWS // _NW


_PW = _B * _WS // _NW
_RPW = _B // _NW
_PWP = _PW + 16


def _sc_gather_body(t_hbm, es_hbm, esg_hbm, mk_hbm,
                    t_v, idx_v, mk_v, ge_v, buf, sem):
    c = lax.axis_index("c")
    s = lax.axis_index("s")
    wid = s * _NC + c
    base = wid * _PW
    row0 = wid * _RPW
    pltpu.sync_copy(t_hbm.at[pl.ds(base, _PW)], t_v.at[pl.ds(0, _PW)])
    lanes = lax.iota(jnp.int32, 16)
    one = jnp.full((16,), 1.0, jnp.float32)
    zero = jnp.full((16,), 0.0, jnp.float32)

    def row_step(r, carry):
        rowbase = (row0 + r) * _N
        for k in range(4):
            o = r * _WS + k * 16
            t = t_v[pl.ds(o, 16)]
            wv = lanes + k * 16
            m = wv < _WS
            a = jnp.where(t > 0, t - 1, _N - 1)
            a = jnp.clip(a, 0, _N - 1)
            plsc.store_scatter(buf, [a], wv, mask=m)
            idx_v[pl.ds(o, 16)] = rowbase + a
        for k in range(4):
            o = r * _WS + k * 16
            t = t_v[pl.ds(o, 16)]
            wv = lanes + k * 16
            m = wv < _WS
            a = jnp.where(t > 0, t - 1, _N - 1)
            a = jnp.clip(a, 0, _N - 1)
            g = plsc.load_gather(buf, [a], mask=m)
            keep = (g == wv) & (t > 0) & m
            mk_v[pl.ds(o, 16)] = jnp.where(keep, one, zero)
        return carry

    lax.fori_loop(0, _RPW, row_step, 0)
    idx_v[pl.ds(_PW, 16)] = jnp.full((16,), 0, jnp.int32)
    pltpu.async_copy(es_hbm.at[idx_v], ge_v, sem).wait()
    pltpu.sync_copy(ge_v.at[pl.ds(0, _PW)], esg_hbm.at[pl.ds(base, _PW)])
    pltpu.sync_copy(mk_v.at[pl.ds(0, _PW)], mk_hbm.at[pl.ds(base, _PW)])


def _sc_gather(t_flat, es_flat):
    f = functools.partial(
        pl.kernel,
        out_type=[jax.ShapeDtypeStruct((_B * _WS,), jnp.float32),
                  jax.ShapeDtypeStruct((_B * _WS,), jnp.float32)],
        mesh=plsc.VectorSubcoreMesh(core_axis_name="c", subcore_axis_name="s"),
        compiler_params=pltpu.CompilerParams(needs_layout_passes=False),
        scratch_types=[
            pltpu.VMEM((_PWP,), jnp.int32),
            pltpu.VMEM((_PWP,), jnp.int32),
            pltpu.VMEM((_PWP,), jnp.float32),
            pltpu.VMEM((_PWP,), jnp.float32),
            pltpu.VMEM((_N,), jnp.int32),
            pltpu.SemaphoreType.DMA,
        ],
    )(_sc_gather_body)
    return f(t_flat, es_flat)


_RD = 512


def _dense_body(es_ref, pr_ref, t_ref, out_ref):
    es = es_ref[...]
    pr = pr_ref[...]
    l1m = jnp.maximum(jnp.log1p(-es), -100.0)
    e2 = jnp.exp(_EXPF * es)
    rmax = jnp.max(pr, axis=1, keepdims=True)
    sexp = jnp.sum(jnp.exp(pr - rmax), axis=1)
    lse = jnp.sum(rmax[:, 0] + jnp.log(sexp))
    t = t_ref[...]
    wio = lax.broadcasted_iota(jnp.int32, t.shape, 1)
    keys = jnp.where(t > 0, wio * 2048 + t, jnp.int32(1 << 30))
    kmin = jnp.min(keys, axis=1, keepdims=True)
    t2 = jnp.where(kmin == (1 << 30), jnp.int32(_N - 1), (kmin & 2047) - 1)
    cio = lax.broadcasted_iota(jnp.int32, pr.shape, 1)
    psel = jnp.sum(jnp.where(cio == t2, pr, 0.0))
    lane = lax.broadcasted_iota(jnp.int32, (1, 1, 128), 2)
    out_ref[...] = jnp.where(lane == 0, jnp.sum(l1m),
                   jnp.where(lane == 1, jnp.sum(e2),
                   jnp.where(lane == 2, lse,
                   jnp.where(lane == 3, psel, 0.0))))


def _dense(es, pr, targets):
    g = _B // _RD
    return pl.pallas_call(
        _dense_body,
        grid=(g,),
        in_specs=[pl.BlockSpec((_RD, _N), lambda i: (i, 0)),
                  pl.BlockSpec((_RD, _N), lambda i: (i, 0)),
                  pl.BlockSpec((_RD, _WS), lambda i: (i, 0))],
        out_specs=pl.BlockSpec((1, 1, 128), lambda i: (i, 0, 0)),
        out_shape=jax.ShapeDtypeStruct((g, 1, 128), jnp.float32),
    )(es, pr, targets)


def _corr_body(ge_ref, mk_ref, dp_ref, o1_ref, o2_ref, o3_ref):
    ge = ge_ref[...]
    mk = mk_ref[...]
    l1m = jnp.maximum(jnp.log1p(-ge), -100.0)
    lp = jnp.maximum(jnp.log(ge), -100.0)
    corr = jnp.sum(mk * (l1m - lp))
    dp = dp_ref[...]
    s_l1m = jnp.sum(dp[:, 0, 0])
    s_e2 = jnp.sum(dp[:, 0, 1])
    s_lse = jnp.sum(dp[:, 0, 2])
    psel = jnp.sum(dp[:, 0, 3])
    m = jnp.float32(_B * _N)
    loss1 = (corr - s_l1m) / m
    o1 = _ALPHA * (loss1 * (s_e2 / m))
    loss2 = (s_lse - psel) / _B
    o1_ref[0] = o1
    o2_ref[0] = loss2
    o3_ref[0] = o1 + loss2


def _corr(esg, mk, dpart):
    sds = jax.ShapeDtypeStruct((1,), jnp.float32)
    return pl.pallas_call(
        _corr_body,
        in_specs=[pl.BlockSpec(memory_space=pltpu.VMEM)] * 3,
        out_specs=[pl.BlockSpec(memory_space=pltpu.SMEM)] * 3,
        out_shape=[sds, sds, sds],
    )(esg, mk, dpart)


def kernel(enable_state, prediction, targets):
    t_flat = targets.reshape(-1)
    es_flat = enable_state.reshape(-1)
    esg, mk = _sc_gather(t_flat, es_flat)
    dense = _dense(enable_state, prediction, targets)
    o1, l2, o3 = _corr(esg, mk, dense)
    return (o1[0], l2[0], o3[0])


# speedup vs baseline: 2.2136x; 1.3575x over previous
import functools

import jax
import jax.numpy as jnp
from jax import lax
from jax.experimental import pallas as pl
from jax.experimental.pallas import tpu as pltpu
from jax.experimental.pallas import tpu_sc as plsc

_B = 16384
_N = 1000
_WS = 50
_ALPHA = 0.1
_EXPF = 2.0

_NC = 2
_NS = 16
_NW = _NC * _NS
_GROWS = _B * _WS // 128
_RW = _GROWS // _NW


_PW = _B * _WS // _NW
_RPW = _B // _NW
_PWP = _PW + 16


_RCH = 16
_NCH = _RPW // _RCH


def _sc_gather_body(t_hbm, es_hbm, esg_hbm, t_v, ge_v, rowbuf, buf, sem):
    c = lax.axis_index("c")
    s = lax.axis_index("s")
    wid = s * _NC + c
    base = wid * _PW
    row0 = wid * _RPW
    pltpu.sync_copy(t_hbm.at[pl.ds(base, _PW)], t_v.at[pl.ds(0, _PW)])
    lanes = lax.iota(jnp.int32, 16)
    half = jnp.full((16,), 0.5, jnp.float32)

    def chunk_step(ch, carry):
        r0 = ch * _RCH
        pltpu.async_copy(es_hbm.at[pl.ds(row0 + r0, _RCH)], rowbuf, sem).wait()

        def row_step(lr, carry2):
            lr_vec = jnp.broadcast_to(lr, (16,)).astype(jnp.int32)
            for k in range(4):
                o = (r0 + lr) * _WS + k * 16
                t = t_v[pl.ds(o, 16)]
                wv = lanes + k * 16
                m = wv < _WS
                a = jnp.where(t > 0, t - 1, _N - 1)
                a = jnp.clip(a, 0, _N - 1)
                plsc.store_scatter(buf, [a], wv, mask=m)
            for k in range(4):
                o = (r0 + lr) * _WS + k * 16
                t = t_v[pl.ds(o, 16)]
                wv = lanes + k * 16
                m = wv < _WS
                a = jnp.where(t > 0, t - 1, _N - 1)
                a = jnp.clip(a, 0, _N - 1)
                g = plsc.load_gather(buf, [a], mask=m)
                keep = (g == wv) & (t > 0) & m
                val = plsc.load_gather(rowbuf, [lr_vec, a], mask=m)
                ge_v[pl.ds(o, 16)] = jnp.where(keep, val, half)
            return carry2

        lax.fori_loop(0, _RCH, row_step, 0)
        return carry

    lax.fori_loop(0, _NCH, chunk_step, 0)
    pltpu.sync_copy(ge_v.at[pl.ds(0, _PW)], esg_hbm.at[pl.ds(base, _PW)])


def _sc_gather(t_flat, es2d):
    f = functools.partial(
        pl.kernel,
        out_type=jax.ShapeDtypeStruct((_B * _WS,), jnp.float32),
        mesh=plsc.VectorSubcoreMesh(core_axis_name="c", subcore_axis_name="s"),
        compiler_params=pltpu.CompilerParams(needs_layout_passes=False),
        scratch_types=[
            pltpu.VMEM((_PWP,), jnp.int32),
            pltpu.VMEM((_PWP,), jnp.float32),
            pltpu.VMEM((_RCH, _N), jnp.float32),
            pltpu.VMEM((_N,), jnp.int32),
            pltpu.SemaphoreType.DMA,
        ],
    )(_sc_gather_body)
    return f(t_flat, es2d)


_RD = 512


def _dense_body(es_ref, pr_ref, t_ref, out_ref):
    es = es_ref[...]
    pr = pr_ref[...]
    l1m = jnp.maximum(jnp.log1p(-es), -100.0)
    e2 = jnp.exp(_EXPF * es)
    rmax = jnp.max(pr, axis=1, keepdims=True)
    sexp = jnp.sum(jnp.exp(pr - rmax), axis=1)
    lse = jnp.sum(rmax[:, 0] + jnp.log(sexp))
    t = t_ref[...]
    wio = lax.broadcasted_iota(jnp.int32, t.shape, 1)
    keys = jnp.where(t > 0, wio * 2048 + t, jnp.int32(1 << 30))
    kmin = jnp.min(keys, axis=1, keepdims=True)
    t2 = jnp.where(kmin == (1 << 30), jnp.int32(_N - 1), (kmin & 2047) - 1)
    cio = lax.broadcasted_iota(jnp.int32, pr.shape, 1)
    psel = jnp.sum(jnp.where(cio == t2, pr, 0.0))
    lane = lax.broadcasted_iota(jnp.int32, (1, 1, 128), 2)
    out_ref[...] = jnp.where(lane == 0, jnp.sum(l1m),
                   jnp.where(lane == 1, jnp.sum(e2),
                   jnp.where(lane == 2, lse,
                   jnp.where(lane == 3, psel, 0.0))))


def _dense(es, pr, targets):
    g = _B // _RD
    return pl.pallas_call(
        _dense_body,
        grid=(g,),
        in_specs=[pl.BlockSpec((_RD, _N), lambda i: (i, 0)),
                  pl.BlockSpec((_RD, _N), lambda i: (i, 0)),
                  pl.BlockSpec((_RD, _WS), lambda i: (i, 0))],
        out_specs=pl.BlockSpec((1, 1, 128), lambda i: (i, 0, 0)),
        out_shape=jax.ShapeDtypeStruct((g, 1, 128), jnp.float32),
    )(es, pr, targets)


def _corr_body(ge_ref, dp_ref, o1_ref, o2_ref, o3_ref):
    ge = ge_ref[...]
    l1m = jnp.maximum(jnp.log1p(-ge), -100.0)
    lp = jnp.maximum(jnp.log(ge), -100.0)
    corr = jnp.sum(l1m - lp)
    dp = dp_ref[...]
    s_l1m = jnp.sum(dp[:, 0, 0])
    s_e2 = jnp.sum(dp[:, 0, 1])
    s_lse = jnp.sum(dp[:, 0, 2])
    psel = jnp.sum(dp[:, 0, 3])
    m = jnp.float32(_B * _N)
    loss1 = (corr - s_l1m) / m
    o1 = _ALPHA * (loss1 * (s_e2 / m))
    loss2 = (s_lse - psel) / _B
    o1_ref[0] = o1
    o2_ref[0] = loss2
    o3_ref[0] = o1 + loss2


def _corr(esg, dpart):
    sds = jax.ShapeDtypeStruct((1,), jnp.float32)
    return pl.pallas_call(
        _corr_body,
        in_specs=[pl.BlockSpec(memory_space=pltpu.VMEM)] * 2,
        out_specs=[pl.BlockSpec(memory_space=pltpu.SMEM)] * 3,
        out_shape=[sds, sds, sds],
    )(esg, dpart)


def kernel(enable_state, prediction, targets):
    t_flat = targets.reshape(-1)
    esg = _sc_gather(t_flat, enable_state)
    dense = _dense(enable_state, prediction, targets)
    o1, l2, o3 = _corr(esg, dense)
    return (o1[0], l2[0], o3[0])
